# baseline (speedup 1.0000x reference)
import functools

import jax
import jax.numpy as jnp
from jax import lax
from jax.experimental import pallas as pl
from jax.experimental.pallas import tpu as pltpu
from jax.experimental.pallas import tpu_sc as plsc

_SEQ_LENS = (2048, 1792, 1536, 1280, 1024, 896, 768, 512)
_D = 1024
_MAX_L = 2048
_NC, _NS = 2, 16
_NW = _NC * _NS
_CH = 32
_NBUF = 3
_PCH = 128
_TOTAL_ROWS = 8 * _MAX_L


def _build_plans():
    segs = []
    for i, L in enumerate(_SEQ_LENS):
        segs.append(("data", i, 0, L))
        if L < _MAX_L:
            segs.append(("pad", i, L, _MAX_L))
    total_w = sum((r1 - r0) * (2 if k == "data" else 1) for k, _, r0, r1 in segs)
    target = total_w / _NW
    plans = [{"data": [], "pad": []} for _ in range(_NW)]
    acc = 0.0
    for kind, i, r0, r1 in segs:
        wpr = 2 if kind == "data" else 1
        r = r0
        while r < r1:
            w = min(_NW - 1, int(acc / target))
            rows = min(8, r1 - r)
            lst = plans[w][kind]
            if lst and lst[-1][0] == i and lst[-1][1] + lst[-1][2] == r:
                lst[-1] = (i, lst[-1][1], lst[-1][2] + rows)
            else:
                lst.append((i, r, rows))
            acc += rows * wpr
            r += rows
    return [( [tuple(x) for x in p["data"]], [tuple(x) for x in p["pad"]] )
            for p in plans]


_PLANS = _build_plans()


def _worker_chunks(w):
    data_segs, _ = _PLANS[w]
    chunks = []
    k = 0
    for i, r0, rows in data_segs:
        r = r0
        while r < r0 + rows:
            n = min(_CH, r0 + rows - r)
            chunks.append((k % _NBUF, i, r, i * _MAX_L + r, n))
            k += 1
            r += n
    return chunks


def _start_read(chunks, c, in_refs, bufs, rd_sems):
    b, i, sr, dr, n = chunks[c]
    return pltpu.async_copy(
        in_refs[i].at[pl.ds(sr, n)], bufs[b].at[pl.ds(0, n)], rd_sems.at[b])


def _emit_worker(w, rd, in_refs, out_ref, bufs, zsh, rd_sems, wr_sems,
                 pad_sem):
    chunks = _worker_chunks(w)
    _, pad_segs = _PLANS[w]
    n_chunks = len(chunks)

    pads = []
    for i, r0, rows in pad_segs:
        r = r0
        while r < r0 + rows:
            n = min(_PCH, r0 + rows - r)
            pads.append(pltpu.async_copy(
                zsh.at[pl.ds(0, n)],
                out_ref.at[pl.ds(i * _MAX_L + r, n)], pad_sem))
            r += n

    wr = [None] * n_chunks
    for c in range(n_chunks):
        b, i, sr, dr, n = chunks[c]
        rd[c].wait()
        wr[c] = pltpu.async_copy(
            bufs[b].at[pl.ds(0, n)], out_ref.at[pl.ds(dr, n)], wr_sems.at[b])
        if c + _NBUF < n_chunks:
            wr[c].wait()
            rd[c + _NBUF] = _start_read(chunks, c + _NBUF, in_refs, bufs,
                                        rd_sems)

    for h in pads:
        h.wait()
    for c in range(max(0, n_chunks - _NBUF), n_chunks):
        wr[c].wait()


def _sc_body(s0, s1, s2, s3, s4, s5, s6, s7, zsrc, out_ref,
             b0, b1, b2, zsh, rd_sems, wr_sems, pad_sem, z_sem):
    in_refs = (s0, s1, s2, s3, s4, s5, s6, s7)
    bufs = (b0, b1, b2)
    cid = lax.axis_index("c")
    sid = lax.axis_index("s")
    wid = sid * _NC + cid

    rd_handles = {}
    for w in range(_NW):
        chunks = _worker_chunks(w)
        rd = [None] * len(chunks)
        rd_handles[w] = rd
        if chunks:
            @pl.when(wid == w)
            def _(w=w, chunks=chunks, rd=rd):
                for c in range(min(_NBUF, len(chunks))):
                    rd[c] = _start_read(chunks, c, in_refs, bufs, rd_sems)

    @pl.when(sid == 0)
    def _():
        pltpu.async_copy(zsrc, zsh, z_sem).wait()

    plsc.subcore_barrier()

    for w in range(_NW):
        @pl.when(wid == w)
        def _(w=w):
            _emit_worker(w, rd_handles[w], in_refs, out_ref, bufs, zsh,
                         rd_sems, wr_sems, pad_sem)


def kernel(seq0, seq1, seq2, seq3, seq4, seq5, seq6, seq7):
    seqs = [s.reshape(s.shape[1], _D) for s in
            (seq0, seq1, seq2, seq3, seq4, seq5, seq6, seq7)]
    zsrc = jnp.zeros((_PCH, _D), jnp.float32)
    mesh = plsc.VectorSubcoreMesh(core_axis_name="c", subcore_axis_name="s")
    run = pl.kernel(
        _sc_body,
        out_type=jax.ShapeDtypeStruct((_TOTAL_ROWS, _D), jnp.float32),
        mesh=mesh,
        scratch_types=[
            pltpu.VMEM((_CH, _D), jnp.float32),
            pltpu.VMEM((_CH, _D), jnp.float32),
            pltpu.VMEM((_CH, _D), jnp.float32),
            pltpu.VMEM_SHARED((_PCH, _D), jnp.float32),
            pltpu.SemaphoreType.DMA((_NBUF,)),
            pltpu.SemaphoreType.DMA((_NBUF,)),
            pltpu.SemaphoreType.DMA,
            pltpu.SemaphoreType.DMA,
        ],
    )
    out = run(*seqs, zsrc)
    return out.reshape(8, _MAX_L, _D)


# speedup vs baseline: 2.2537x; 2.2537x over previous
import jax
import jax.numpy as jnp
from jax.experimental import pallas as pl
from jax.experimental.pallas import tpu as pltpu

_SEQ_LENS = (2048, 1792, 1536, 1280, 1024, 896, 768, 512)
_D = 1024
_MAX_L = 2048
_MAX_PAD = _MAX_L - min(_SEQ_LENS)
_CHUNK = 512
_N_CHUNKS = sum(-(-L // _CHUNK) for L in _SEQ_LENS)


def _body(*refs):
    in_refs = refs[:8]
    out_ref = refs[8]
    bufs = refs[9:17]
    zero_ref = refs[17]
    in_sems = refs[18]
    out_sems = refs[19]
    pad_sems = refs[20]

    chunks = []
    for t in range(_MAX_L // _CHUNK):
        for i, L in enumerate(_SEQ_LENS):
            r0 = t * _CHUNK
            rows = min(_CHUNK, L - r0)
            if rows > 0:
                chunks.append((i, r0, rows))

    in_copies = []
    for ci, (i, r0, rows) in enumerate(chunks):
        c = pltpu.make_async_copy(
            in_refs[i].at[:, pl.ds(r0, rows), :],
            bufs[i].at[:, pl.ds(r0, rows), :],
            in_sems.at[ci],
        )
        c.start()
        in_copies.append(c)

    zero_ref[...] = jnp.zeros(zero_ref.shape, zero_ref.dtype)

    pad_copies = []
    for i, L in enumerate(_SEQ_LENS):
        pad = _MAX_L - L
        if pad:
            c = pltpu.make_async_copy(
                zero_ref.at[:, pl.ds(0, pad), :],
                out_ref.at[pl.ds(i, 1), pl.ds(L, pad), :],
                pad_sems.at[i],
            )
            c.start()
            pad_copies.append(c)

    out_copies = []
    for ci, (i, r0, rows) in enumerate(chunks):
        in_copies[ci].wait()
        c = pltpu.make_async_copy(
            bufs[i].at[:, pl.ds(r0, rows), :],
            out_ref.at[pl.ds(i, 1), pl.ds(r0, rows), :],
            out_sems.at[ci],
        )
        c.start()
        out_copies.append(c)

    for c in out_copies:
        c.wait()
    for c in pad_copies:
        c.wait()


def kernel(seq0, seq1, seq2, seq3, seq4, seq5, seq6, seq7):
    seqs = (seq0, seq1, seq2, seq3, seq4, seq5, seq6, seq7)
    out_shape = jax.ShapeDtypeStruct((8, _MAX_L, _D), seq0.dtype)
    return pl.pallas_call(
        _body,
        in_specs=[pl.BlockSpec(memory_space=pl.ANY)] * 8,
        out_specs=pl.BlockSpec(memory_space=pl.ANY),
        out_shape=out_shape,
        scratch_shapes=(
            [pltpu.VMEM((1, L, _D), jnp.float32) for L in _SEQ_LENS]
            + [
                pltpu.VMEM((1, _MAX_PAD, _D), jnp.float32),
                pltpu.SemaphoreType.DMA((_N_CHUNKS,)),
                pltpu.SemaphoreType.DMA((_N_CHUNKS,)),
                pltpu.SemaphoreType.DMA((8,)),
            ]
        ),
    )(*seqs)
